# baseline (speedup 1.0000x reference)
import functools

import jax
import jax.numpy as jnp
from jax.experimental import pallas as pl
from jax.experimental.pallas import tpu as pltpu
from jax.experimental.pallas import tpu_sc as plsc

E = 8
CAP = 512
LOOK_BACK = 5
LOOK_AHEAD = 5
PAD = 5
CHUNK = 1024


def _router_body(e_ref, x_ref, rwe_ref, rwx_ref,
                 dstw_ref, dstr_ref, gatek_ref, keep_ref):
    N = e_ref.shape[0]
    logits = (
        jax.lax.dot_general(e_ref[...], rwe_ref[...], (((1,), (0,)), ((), ())),
                            preferred_element_type=jnp.float32)
        + jax.lax.dot_general(x_ref[...], rwx_ref[...], (((1,), (0,)), ((), ())),
                              preferred_element_type=jnp.float32)
    )
    lmax = jnp.max(logits, axis=-1, keepdims=True)
    denom = jnp.sum(jnp.exp(logits - lmax), axis=-1, keepdims=True)
    gate = 1.0 / denom
    iota_e = jax.lax.broadcasted_iota(jnp.int32, (N, E), 1)
    is_max = logits == lmax
    idx = jnp.min(jnp.where(is_max, iota_e, E), axis=-1, keepdims=True)
    oh = (iota_e == idx).astype(jnp.float32)
    tri = (jax.lax.broadcasted_iota(jnp.int32, (CHUNK, CHUNK), 0)
           >= jax.lax.broadcasted_iota(jnp.int32, (CHUNK, CHUNK), 1)
           ).astype(jnp.float32)
    carry = jnp.zeros((1, E), jnp.float32)
    pos_parts = []
    for i in range(N // CHUNK):
        ohi = jax.lax.slice(oh, (i * CHUNK, 0), ((i + 1) * CHUNK, E))
        ci = jax.lax.dot_general(tri, ohi, (((1,), (0,)), ((), ())),
                                 preferred_element_type=jnp.float32) + carry
        carry = jax.lax.slice(ci, (CHUNK - 1, 0), (CHUNK, E))
        pos_parts.append(jnp.sum(ci * ohi, axis=-1, keepdims=True) - 1.0)
    pos = jnp.concatenate(pos_parts, axis=0).astype(jnp.int32)
    keep = pos < CAP
    tok = jax.lax.broadcasted_iota(jnp.int32, (N, 1), 0)
    flat = idx * CAP + pos
    dstw_ref[...] = jnp.where(keep, flat, E * CAP + tok)
    dstr_ref[...] = jnp.where(keep, flat, 0)
    gatek_ref[...] = jnp.where(keep, gate, 0.0)
    keep_ref[...] = keep.astype(jnp.float32)


def _router_indices(e2d, x2d, rwe, rwx):
    N = x2d.shape[0]
    return pl.pallas_call(
        _router_body,
        out_shape=(
            jax.ShapeDtypeStruct((N, 1), jnp.int32),
            jax.ShapeDtypeStruct((N, 1), jnp.int32),
            jax.ShapeDtypeStruct((N, 1), jnp.float32),
            jax.ShapeDtypeStruct((N, 1), jnp.float32),
        ),
    )(e2d, x2d, rwe, rwx)


def _expert_body2(buf_ref, w1_ref, b1_ref, w2_ref, m_ref):
    h = jax.lax.dot_general(buf_ref[...], w1_ref[0], (((1,), (0,)), ((), ())),
                            preferred_element_type=jnp.float32)
    h = jnp.maximum(h + b1_ref[0], 0.0)
    m_ref[...] = jax.lax.dot_general(
        h, w2_ref[0], (((1,), (0,)), ((), ())),
        preferred_element_type=jnp.float32).astype(jnp.bfloat16)


def _experts(buf_ext, w1, b1, w2):
    D_HID = w1.shape[-1]
    D = w2.shape[-1]
    return pl.pallas_call(
        _expert_body2,
        grid=(E,),
        in_specs=[
            pl.BlockSpec((CAP, D), lambda e: (e, 0)),
            pl.BlockSpec((1, D, D_HID), lambda e: (e, 0, 0)),
            pl.BlockSpec((1, 1, D_HID), lambda e: (e, 0, 0)),
            pl.BlockSpec((1, D_HID, D), lambda e: (e, 0, 0)),
        ],
        out_specs=pl.BlockSpec((CAP, D), lambda e: (e, 0)),
        out_shape=jax.ShapeDtypeStruct((E * CAP, D), jnp.bfloat16),
    )(buf_ext, w1, b1.reshape(E, 1, D_HID), w2)


def _sc_dispatch(x2d, dstw):
    NTOK, D = x2d.shape
    info = plsc.get_sparse_core_info()
    nc, ns = info.num_cores, info.num_subcores
    per = NTOK // (nc * ns)
    mesh = plsc.VectorSubcoreMesh(core_axis_name="c", subcore_axis_name="s")

    @functools.partial(
        pl.kernel, mesh=mesh,
        out_type=jax.ShapeDtypeStruct((E * CAP + NTOK, D), jnp.float32),
        scratch_types=[
            pltpu.VMEM((per,), jnp.int32),
            pltpu.VMEM((per, D), jnp.float32),
            pltpu.SemaphoreType.DMA,
        ],
    )
    def k(x_hbm, dw_hbm, buf_hbm, idx_v, rows_v, sem):
        wid = jax.lax.axis_index("s") * nc + jax.lax.axis_index("c")
        base = wid * per
        pltpu.sync_copy(dw_hbm.at[pl.ds(base, per)], idx_v)
        pltpu.sync_copy(x_hbm.at[pl.ds(base, per)], rows_v)
        pltpu.async_copy(rows_v, buf_hbm.at[idx_v], sem).wait()

    return k(x2d, dstw)


def _sc_combine(m2d, dstr):
    NTOK = dstr.shape[0]
    D = m2d.shape[1]
    info = plsc.get_sparse_core_info()
    nc, ns = info.num_cores, info.num_subcores
    per = NTOK // (nc * ns)
    mesh = plsc.VectorSubcoreMesh(core_axis_name="c", subcore_axis_name="s")

    @functools.partial(
        pl.kernel, mesh=mesh,
        out_type=jax.ShapeDtypeStruct((NTOK, D), m2d.dtype),
        scratch_types=[
            pltpu.VMEM((per,), jnp.int32),
            pltpu.VMEM((per, D), m2d.dtype),
            pltpu.SemaphoreType.DMA,
        ],
    )
    def k(m_hbm, dr_hbm, y_hbm, idx_v, rows_v, sem):
        wid = jax.lax.axis_index("s") * nc + jax.lax.axis_index("c")
        base = wid * per
        pltpu.sync_copy(dr_hbm.at[pl.ds(base, per)], idx_v)
        pltpu.async_copy(m_hbm.at[idx_v], rows_v, sem).wait()
        pltpu.sync_copy(rows_v, y_hbm.at[pl.ds(base, per)])

    return k(m2d, dstr)


def _fsmn_body(y_ref, gk_ref, kp_ref, x_ref, mask_ref, lf_ref, cf_ref, rf_ref,
               out_ref):
    T = x_ref.shape[1]
    D = x_ref.shape[2]
    p = jnp.where(kp_ref[0] > 0.0,
                  y_ref[0].astype(jnp.float32) * gk_ref[0], 0.0)
    z = jnp.zeros((PAD, D), jnp.float32)
    pz = jnp.concatenate([z, p, z], axis=0)
    acc = p * cf_ref[0]
    for i in range(1, LOOK_BACK + 1):
        s = PAD - i
        acc = acc + jax.lax.slice(pz, (s, 0), (s + T, D)) * lf_ref[i - 1]
    for j in range(1, LOOK_AHEAD + 1):
        s = PAD + j
        acc = acc + jax.lax.slice(pz, (s, 0), (s + T, D)) * rf_ref[j - 1]
    out_ref[0] = (acc + x_ref[0]) * mask_ref[0]


def _fsmn(y3, gk3, kp3, inputs, mask3, lf, cf, rf):
    Bq, Tq, D = inputs.shape
    DC = D // 2
    return pl.pallas_call(
        _fsmn_body,
        grid=(Bq, 2),
        in_specs=[
            pl.BlockSpec((1, Tq, DC), lambda b, d: (b, 0, d)),
            pl.BlockSpec((1, Tq, 1), lambda b, d: (b, 0, 0)),
            pl.BlockSpec((1, Tq, 1), lambda b, d: (b, 0, 0)),
            pl.BlockSpec((1, Tq, DC), lambda b, d: (b, 0, d)),
            pl.BlockSpec((1, Tq, 1), lambda b, d: (b, 0, 0)),
            pl.BlockSpec((LOOK_BACK, DC), lambda b, d: (0, d)),
            pl.BlockSpec((1, DC), lambda b, d: (0, d)),
            pl.BlockSpec((LOOK_AHEAD, DC), lambda b, d: (0, d)),
        ],
        out_specs=pl.BlockSpec((1, Tq, DC), lambda b, d: (b, 0, d)),
        out_shape=jax.ShapeDtypeStruct((Bq, Tq, D), jnp.float32),
    )(y3, gk3, kp3, inputs, mask3, lf, cf, rf)


def kernel(inputs, embed, seq_len, is_training, w1, b1, w2,
           left_factor, cur_factor, right_factor, router_w):
    Bq, Tq, Din = inputs.shape
    N = Bq * Tq
    D = w2.shape[-1]
    x2d = inputs.reshape(N, Din)
    e2d = embed.reshape(N, embed.shape[-1])
    rwe = router_w[:embed.shape[-1]]
    rwx = router_w[embed.shape[-1]:]

    dstw, dstr, gatek, keepf = _router_indices(e2d, x2d, rwe, rwx)
    dstw = dstw[:, 0]
    dstr = dstr[:, 0]

    buf_ext = _sc_dispatch(x2d, dstw)

    m = _experts(buf_ext, w1, b1, w2)

    m_i = jax.lax.bitcast_convert_type(m.reshape(E * CAP, D // 2, 2), jnp.int32)
    y_i = _sc_combine(m_i, dstr)
    y3 = jax.lax.bitcast_convert_type(y_i, jnp.bfloat16).reshape(Bq, Tq, D)

    mask3 = (jnp.arange(Tq)[None, :, None] < seq_len[:, None, None]).astype(jnp.float32)
    return _fsmn(y3, gatek.reshape(Bq, Tq, 1), keepf.reshape(Bq, Tq, 1),
                 inputs, mask3, left_factor, cur_factor, right_factor)


# speedup vs baseline: 1.9748x; 1.9748x over previous
import functools

import jax
import jax.numpy as jnp
from jax.experimental import pallas as pl
from jax.experimental.pallas import tpu as pltpu
from jax.experimental.pallas import tpu_sc as plsc

E = 8
CAP = 512
LOOK_BACK = 5
LOOK_AHEAD = 5
PAD = 5
CHUNK = 1024


def _router_body(e_ref, x_ref, rwe_ref, rwx_ref,
                 dstw_ref, dstr_ref, gatek_ref):
    N = e_ref.shape[0]
    logits = (
        jax.lax.dot_general(e_ref[...], rwe_ref[...], (((1,), (0,)), ((), ())),
                            preferred_element_type=jnp.float32)
        + jax.lax.dot_general(x_ref[...], rwx_ref[...], (((1,), (0,)), ((), ())),
                              preferred_element_type=jnp.float32)
    )
    lmax = jnp.max(logits, axis=-1, keepdims=True)
    denom = jnp.sum(jnp.exp(logits - lmax), axis=-1, keepdims=True)
    gate = 1.0 / denom
    iota_e = jax.lax.broadcasted_iota(jnp.int32, (N, E), 1)
    is_max = logits == lmax
    idx = jnp.min(jnp.where(is_max, iota_e, E), axis=-1, keepdims=True)
    oh = (iota_e == idx).astype(jnp.float32)
    tri = (jax.lax.broadcasted_iota(jnp.int32, (CHUNK, CHUNK), 0)
           >= jax.lax.broadcasted_iota(jnp.int32, (CHUNK, CHUNK), 1)
           ).astype(jnp.float32)
    carry = jnp.zeros((1, E), jnp.float32)
    pos_parts = []
    for i in range(N // CHUNK):
        ohi = jax.lax.slice(oh, (i * CHUNK, 0), ((i + 1) * CHUNK, E))
        ci = jax.lax.dot_general(tri, ohi, (((1,), (0,)), ((), ())),
                                 preferred_element_type=jnp.float32) + carry
        carry = jax.lax.slice(ci, (CHUNK - 1, 0), (CHUNK, E))
        pos_parts.append(jnp.sum(ci * ohi, axis=-1, keepdims=True) - 1.0)
    pos = jnp.concatenate(pos_parts, axis=0).astype(jnp.int32)
    keep = pos < CAP
    tok = jax.lax.broadcasted_iota(jnp.int32, (N, 1), 0)
    flat = idx * CAP + pos
    nrow = N // 128
    dstw_ref[...] = jnp.where(keep, flat, E * CAP + tok).reshape(nrow, 128)
    dstr_ref[...] = jnp.where(keep, flat, 0).reshape(nrow, 128)
    gatek_ref[...] = jnp.where(keep, gate, 0.0)


def _router_indices(e2d, x2d, rwe, rwx):
    N = x2d.shape[0]
    return pl.pallas_call(
        _router_body,
        out_shape=(
            jax.ShapeDtypeStruct((N // 128, 128), jnp.int32),
            jax.ShapeDtypeStruct((N // 128, 128), jnp.int32),
            jax.ShapeDtypeStruct((N, 1), jnp.float32),
        ),
    )(e2d, x2d, rwe, rwx)


def _expert_body2(buf_ref, w1_ref, b1_ref, w2_ref, m_ref):
    h = jax.lax.dot_general(buf_ref[...], w1_ref[0], (((1,), (0,)), ((), ())),
                            preferred_element_type=jnp.float32)
    h = jnp.maximum(h + b1_ref[0], 0.0)
    m_ref[...] = jax.lax.dot_general(h, w2_ref[0], (((1,), (0,)), ((), ())),
                                     preferred_element_type=jnp.float32)


def _experts(buf_ext, w1, b1, w2):
    D_HID = w1.shape[-1]
    D = w2.shape[-1]
    return pl.pallas_call(
        _expert_body2,
        grid=(E,),
        in_specs=[
            pl.BlockSpec((CAP, D), lambda e: (e, 0)),
            pl.BlockSpec((1, D, D_HID), lambda e: (e, 0, 0)),
            pl.BlockSpec((1, 1, D_HID), lambda e: (e, 0, 0)),
            pl.BlockSpec((1, D_HID, D), lambda e: (e, 0, 0)),
        ],
        out_specs=pl.BlockSpec((CAP, D), lambda e: (e, 0)),
        out_shape=jax.ShapeDtypeStruct((E * CAP, D), jnp.float32),
    )(buf_ext, w1, b1.reshape(E, 1, D_HID), w2)


def _sc_dispatch(x2d, dstw):
    NTOK, D = x2d.shape
    info = plsc.get_sparse_core_info()
    nc, ns = info.num_cores, info.num_subcores
    per = NTOK // (nc * ns)
    mesh = plsc.VectorSubcoreMesh(core_axis_name="c", subcore_axis_name="s")

    @functools.partial(
        pl.kernel, mesh=mesh,
        out_type=jax.ShapeDtypeStruct((E * CAP + NTOK, D), jnp.float32),
        scratch_types=[
            pltpu.VMEM((per,), jnp.int32),
            pltpu.VMEM((per, D), jnp.float32),
            pltpu.SemaphoreType.DMA,
        ],
    )
    def k(x_hbm, dw_hbm, buf_hbm, idx_v, rows_v, sem):
        wid = jax.lax.axis_index("s") * nc + jax.lax.axis_index("c")
        base = wid * per
        pltpu.sync_copy(dw_hbm.at[pl.ds(base, per)], idx_v)
        pltpu.sync_copy(x_hbm.at[pl.ds(base, per)], rows_v)
        pltpu.async_copy(rows_v, buf_hbm.at[idx_v], sem).wait()

    return k(x2d, dstw)


def _sc_combine(m2d, dstr):
    NTOK = dstr.shape[0]
    D = m2d.shape[1]
    info = plsc.get_sparse_core_info()
    nc, ns = info.num_cores, info.num_subcores
    per = NTOK // (nc * ns)
    mesh = plsc.VectorSubcoreMesh(core_axis_name="c", subcore_axis_name="s")

    @functools.partial(
        pl.kernel, mesh=mesh,
        out_type=jax.ShapeDtypeStruct((NTOK, D), jnp.float32),
        scratch_types=[
            pltpu.VMEM((per,), jnp.int32),
            pltpu.VMEM((per, D), jnp.float32),
            pltpu.SemaphoreType.DMA,
        ],
    )
    def k(m_hbm, dr_hbm, y_hbm, idx_v, rows_v, sem):
        wid = jax.lax.axis_index("s") * nc + jax.lax.axis_index("c")
        base = wid * per
        pltpu.sync_copy(dr_hbm.at[pl.ds(base, per)], idx_v)
        pltpu.async_copy(m_hbm.at[idx_v], rows_v, sem).wait()
        pltpu.sync_copy(rows_v, y_hbm.at[pl.ds(base, per)])

    return k(m2d, dstr)


def _fsmn_body(y_ref, gk_ref, x_ref, mask_ref, lf_ref, cf_ref, rf_ref,
               out_ref):
    T = x_ref.shape[1]
    D = x_ref.shape[2]
    gk = gk_ref[0]
    p = jnp.where(gk > 0.0, y_ref[0] * gk, 0.0)
    z = jnp.zeros((PAD, D), jnp.float32)
    pz = jnp.concatenate([z, p, z], axis=0)
    acc = p * cf_ref[0]
    for i in range(1, LOOK_BACK + 1):
        s = PAD - i
        acc = acc + jax.lax.slice(pz, (s, 0), (s + T, D)) * lf_ref[i - 1]
    for j in range(1, LOOK_AHEAD + 1):
        s = PAD + j
        acc = acc + jax.lax.slice(pz, (s, 0), (s + T, D)) * rf_ref[j - 1]
    out_ref[0] = (acc + x_ref[0]) * mask_ref[0]


def _fsmn(y3, gk3, inputs, mask3, lf, cf, rf):
    Bq, Tq, D = inputs.shape
    DC = D // 2
    return pl.pallas_call(
        _fsmn_body,
        grid=(Bq, 2),
        in_specs=[
            pl.BlockSpec((1, Tq, DC), lambda b, d: (b, 0, d)),
            pl.BlockSpec((1, Tq, 1), lambda b, d: (b, 0, 0)),
            pl.BlockSpec((1, Tq, DC), lambda b, d: (b, 0, d)),
            pl.BlockSpec((1, Tq, 1), lambda b, d: (b, 0, 0)),
            pl.BlockSpec((LOOK_BACK, DC), lambda b, d: (0, d)),
            pl.BlockSpec((1, DC), lambda b, d: (0, d)),
            pl.BlockSpec((LOOK_AHEAD, DC), lambda b, d: (0, d)),
        ],
        out_specs=pl.BlockSpec((1, Tq, DC), lambda b, d: (b, 0, d)),
        out_shape=jax.ShapeDtypeStruct((Bq, Tq, D), jnp.float32),
    )(y3, gk3, inputs, mask3, lf, cf, rf)


def kernel(inputs, embed, seq_len, is_training, w1, b1, w2,
           left_factor, cur_factor, right_factor, router_w):
    Bq, Tq, Din = inputs.shape
    N = Bq * Tq
    D = w2.shape[-1]
    x2d = inputs.reshape(N, Din)
    e2d = embed.reshape(N, embed.shape[-1])
    rwe = router_w[:embed.shape[-1]]
    rwx = router_w[embed.shape[-1]:]

    dstw, dstr, gatek = _router_indices(e2d, x2d, rwe, rwx)
    dstw = dstw.reshape(N)
    dstr = dstr.reshape(N)

    buf_ext = _sc_dispatch(x2d, dstw)

    m = _experts(buf_ext, w1, b1, w2)

    y3 = _sc_combine(m, dstr).reshape(Bq, Tq, D)

    mask3 = (jnp.arange(Tq)[None, :, None] < seq_len[:, None, None]).astype(jnp.float32)
    return _fsmn(y3, gatek.reshape(Bq, Tq, 1),
                 inputs, mask3, left_factor, cur_factor, right_factor)
